# Initial kernel scaffold; baseline (speedup 1.0000x reference)
import jax
import jax.numpy as jnp
from jax.experimental import pallas as pl


def kernel(x, weight, bias, psi_ker_idx, psi_row_idx, psi_col_idx, psi_vals):
    raise NotImplementedError("write your pallas kernel here")


# speedup vs baseline: 79.6534x; 79.6534x over previous
import jax
import jax.numpy as jnp
from jax.experimental import pallas as pl
from jax.experimental.pallas import tpu as pltpu

NLAT_IN, NLON_IN = 46, 90
NLAT_OUT, NLON_OUT = 92, 180
K = 9
C_IN = 128
C_OUT = 64
NNZ = 30000
NROWS = K * NLAT_IN
NOUT_ROWS = NLAT_OUT * 2 * 90
NCHUNK = 20
CHUNK = NNZ // NCHUNK


def _body(x_ref, w_ref, ktin_ref, s_ref, db_ref, val_ref, out_ref, xk2, acc):
    g = pl.program_id(0)

    @pl.when(g < K)
    def _matmul():
        xkk = jax.lax.dot_general(
            x_ref[...], w_ref[0],
            (((0,), (0,)), ((), ())),
            preferred_element_type=jnp.float32,
        )
        xkk = xkk.reshape(NLAT_IN, NLON_IN, C_OUT)
        row0 = g * NLAT_IN
        xk2[pl.ds(row0, NLAT_IN), pl.ds(0, NLON_IN), :] = xkk
        xk2[pl.ds(row0, NLAT_IN), pl.ds(NLON_IN, NLON_IN), :] = xkk

    @pl.when(g == K)
    def _zero():
        acc[...] = jnp.zeros_like(acc)

    @pl.when(g >= K)
    def _scatter():
        def step(e, carry):
            ktin = ktin_ref[0, 0, e]
            s = s_ref[0, 0, e]
            db = db_ref[0, 0, e]
            v = val_ref[0, 0, e]
            src = xk2[ktin, pl.ds(s, NLON_IN), :]
            cur = acc[pl.ds(db, NLON_IN), :]
            acc[pl.ds(db, NLON_IN), :] = cur + v * src
            return carry

        jax.lax.fori_loop(0, CHUNK, step, 0)

    @pl.when(g == K + NCHUNK - 1)
    def _flush():
        out_ref[...] = acc[...]


def kernel(x, weight, bias, psi_ker_idx, psi_row_idx, psi_col_idx, psi_vals):
    ker = psi_ker_idx.astype(jnp.int32)
    tin = psi_row_idx.astype(jnp.int32)
    col = psi_col_idx.astype(jnp.int32)

    tout = col // NLON_OUT
    pout = (NLON_OUT - 1) - (col % NLON_OUT)
    m = pout // 2
    p = 1 - (pout % 2)
    s = (m + 1) % NLON_IN
    ktin = ker * NLAT_IN + tin
    db = (tout * 2 + p) * NLON_IN

    ktin_c = ktin.reshape(NCHUNK, 1, CHUNK)
    s_c = s.reshape(NCHUNK, 1, CHUNK)
    db_c = db.reshape(NCHUNK, 1, CHUNK)
    val_c = psi_vals.reshape(NCHUNK, 1, CHUNK)

    xf = x.reshape(C_IN, NLAT_IN * NLON_IN)
    wt = weight.transpose(2, 1, 0)

    grid = (K + NCHUNK,)
    acc_flat = pl.pallas_call(
        _body,
        grid=grid,
        in_specs=[
            pl.BlockSpec((C_IN, NLAT_IN * NLON_IN), lambda g: (0, 0)),
            pl.BlockSpec((1, C_IN, C_OUT), lambda g: (jnp.minimum(g, K - 1), 0, 0)),
            pl.BlockSpec((1, 1, CHUNK), lambda g: (jnp.clip(g - K, 0, NCHUNK - 1), 0, 0),
                         memory_space=pltpu.SMEM),
            pl.BlockSpec((1, 1, CHUNK), lambda g: (jnp.clip(g - K, 0, NCHUNK - 1), 0, 0),
                         memory_space=pltpu.SMEM),
            pl.BlockSpec((1, 1, CHUNK), lambda g: (jnp.clip(g - K, 0, NCHUNK - 1), 0, 0),
                         memory_space=pltpu.SMEM),
            pl.BlockSpec((1, 1, CHUNK), lambda g: (jnp.clip(g - K, 0, NCHUNK - 1), 0, 0),
                         memory_space=pltpu.SMEM),
        ],
        out_specs=pl.BlockSpec((NOUT_ROWS, C_OUT), lambda g: (0, 0)),
        out_shape=jax.ShapeDtypeStruct((NOUT_ROWS, C_OUT), jnp.float32),
        scratch_shapes=[
            pltpu.VMEM((NROWS, 2 * NLON_IN, C_OUT), jnp.float32),
            pltpu.VMEM((NOUT_ROWS, C_OUT), jnp.float32),
        ],
    )(xf, wt, ktin_c, s_c, db_c, val_c)

    out = acc_flat.reshape(NLAT_OUT, 2, NLON_IN, C_OUT)
    out = out.transpose(3, 0, 2, 1).reshape(1, C_OUT, NLAT_OUT, NLON_OUT)
    return out + bias.reshape(1, -1, 1, 1)
